# Initial kernel scaffold; baseline (speedup 1.0000x reference)
import jax
import jax.numpy as jnp
from jax.experimental import pallas as pl


def kernel(pos, rgb, opa, quat, scale, rot, tran, k):
    raise NotImplementedError("write your pallas kernel here")


# speedup vs baseline: 8.1830x; 8.1830x over previous
import jax
import jax.numpy as jnp
from jax.experimental import pallas as pl

NEAR = 0.1
TOPK = 65536
LANES = 8192


def _feats_kernel(pk_ref, cam_ref, out_ref):
    px = pk_ref[0, :]
    py = pk_ref[1, :]
    pz = pk_ref[2, :]
    r00 = cam_ref[0, 0]; r01 = cam_ref[0, 1]; r02 = cam_ref[0, 2]
    r10 = cam_ref[0, 3]; r11 = cam_ref[0, 4]; r12 = cam_ref[0, 5]
    r20 = cam_ref[0, 6]; r21 = cam_ref[0, 7]; r22 = cam_ref[0, 8]
    t0 = cam_ref[0, 9]; t1 = cam_ref[0, 10]; t2 = cam_ref[0, 11]

    x = px * r00 + py * r01 + pz * r02 + t0
    y = px * r10 + py * r11 + pz * r12 + t1
    z = px * r20 + py * r21 + pz * r22 + t2
    z = jnp.where(z > NEAR, z, NEAR)
    l = jnp.sqrt(x * x + y * y + z * z) + 1e-8
    inv_z = 1.0 / z
    inv_l = 1.0 / l

    out_ref[0, :] = x * inv_z
    out_ref[1, :] = y * inv_z
    out_ref[2, :] = l
    out_ref[3, :] = pk_ref[3, :]
    out_ref[4, :] = pk_ref[4, :]
    out_ref[5, :] = pk_ref[5, :]

    qw = pk_ref[7, :]
    qx = pk_ref[8, :]
    qy = pk_ref[9, :]
    qz = pk_ref[10, :]
    qnorm = jnp.sqrt(qw * qw + qx * qx + qy * qy + qz * qz) + 1e-8
    qn = 1.0 / qnorm
    qw = qw * qn; qx = qx * qn; qy = qy * qn; qz = qz * qn

    R00 = 1 - 2 * (qy * qy + qz * qz)
    R01 = 2 * (qx * qy - qw * qz)
    R02 = 2 * (qx * qz + qw * qy)
    R10 = 2 * (qx * qy + qw * qz)
    R11 = 1 - 2 * (qx * qx + qz * qz)
    R12 = 2 * (qy * qz - qw * qx)
    R20 = 2 * (qx * qz - qw * qy)
    R21 = 2 * (qy * qz + qw * qx)
    R22 = 1 - 2 * (qx * qx + qy * qy)

    s0 = jnp.abs(pk_ref[11, :]) + 1e-4
    s1 = jnp.abs(pk_ref[12, :]) + 1e-4
    s2 = jnp.abs(pk_ref[13, :]) + 1e-4

    a0 = R00 * s0; a1 = R01 * s1; a2 = R02 * s2
    b0 = R10 * s0; b1 = R11 * s1; b2 = R12 * s2
    c0 = R20 * s0; c1 = R21 * s1; c2 = R22 * s2
    C00 = a0 * a0 + a1 * a1 + a2 * a2
    C01 = a0 * b0 + a1 * b1 + a2 * b2
    C02 = a0 * c0 + a1 * c1 + a2 * c2
    C11 = b0 * b0 + b1 * b1 + b2 * b2
    C12 = b0 * c0 + b1 * c1 + b2 * c2
    C22 = c0 * c0 + c1 * c1 + c2 * c2

    inv_z2 = inv_z * inv_z
    j00 = inv_z; j02 = -x * inv_z2
    j11 = inv_z; j12 = -y * inv_z2
    w00 = j00 * r00 + j02 * r20
    w01 = j00 * r01 + j02 * r21
    w02 = j00 * r02 + j02 * r22
    w10 = j11 * r10 + j12 * r20
    w11 = j11 * r11 + j12 * r21
    w12 = j11 * r12 + j12 * r22

    m00 = w00 * C00 + w01 * C01 + w02 * C02
    m01 = w00 * C01 + w01 * C11 + w02 * C12
    m02 = w00 * C02 + w01 * C12 + w02 * C22
    m10 = w10 * C00 + w11 * C01 + w12 * C02
    m11 = w10 * C01 + w11 * C11 + w12 * C12
    m12 = w10 * C02 + w11 * C12 + w12 * C22
    v00 = m00 * w00 + m01 * w01 + m02 * w02
    v01 = m00 * w10 + m01 * w11 + m02 * w12
    v10 = m10 * w00 + m11 * w01 + m12 * w02
    v11 = m10 * w10 + m11 * w11 + m12 * w12

    out_ref[6, :] = v00
    out_ref[7, :] = v01
    out_ref[8, :] = v10
    out_ref[9, :] = v11

    sc = pk_ref[6, :]
    out_ref[10, :] = sc
    out_ref[11, :] = jnp.zeros_like(sc)
    out_ref[12, :] = jnp.zeros_like(sc)
    out_ref[13, :] = jnp.zeros_like(sc)
    out_ref[14, :] = jnp.zeros_like(sc)
    out_ref[15, :] = jnp.zeros_like(sc)


def _dense_feats(pk, cam, npad):
    grid = npad // LANES
    return pl.pallas_call(
        _feats_kernel,
        grid=(grid,),
        in_specs=[
            pl.BlockSpec((16, LANES), lambda i: (0, i)),
            pl.BlockSpec((8, 128), lambda i: (0, 0)),
        ],
        out_specs=pl.BlockSpec((16, LANES), lambda i: (0, i)),
        out_shape=jax.ShapeDtypeStruct((16, npad), jnp.float32),
    )(pk, cam)


def kernel(pos, rgb, opa, quat, scale, rot, tran, k):
    n = pos.shape[0]
    npad = ((n + LANES - 1) // LANES) * LANES
    scores_in = jax.nn.sigmoid(opa)
    pk = jnp.concatenate(
        [pos.T, rgb.T, scores_in[None, :], quat.T, scale.T,
         jnp.zeros((2, n), jnp.float32)], axis=0)
    pk = jnp.pad(pk, ((0, 0), (0, npad - n)))
    cam = jnp.zeros((8, 128), jnp.float32)
    cam = cam.at[0, :9].set(rot.reshape(-1))
    cam = cam.at[0, 9:12].set(tran)

    feats = _dense_feats(pk, cam, npad)

    scores = feats[10, :n]
    topv, topi = jax.lax.top_k(scores, TOPK)
    g = feats[:11, :].T[topi]
    out = jnp.concatenate([g[:, :10], topv[:, None]], axis=-1)
    return out
